# Initial kernel scaffold; baseline (speedup 1.0000x reference)
import jax
import jax.numpy as jnp
from jax.experimental import pallas as pl


def kernel(xs, ilens, codebook):
    raise NotImplementedError("write your pallas kernel here")


# speedup vs baseline: 1.0876x; 1.0876x over previous
import jax
import jax.numpy as jnp
from jax.experimental import pallas as pl
from jax.experimental.pallas import tpu as pltpu

_NT = 512
_KT = 1024


def _vq_loss_kernel(maxlen_ref, x_ref, c_ref, out_ref, acc_ref):
    i = pl.program_id(0)
    j = pl.program_id(1)
    nk = pl.num_programs(1)
    max_ilen = maxlen_ref[0]
    t_dim = maxlen_ref[1]

    t0 = (i * _NT) % t_dim
    tile_active = t0 < max_ilen

    @pl.when(jnp.logical_and(i == 0, j == 0))
    def _init_out():
        out_ref[0, 0] = 0.0

    @pl.when(tile_active)
    def _compute():
        @pl.when(j == 0)
        def _init_acc():
            acc_ref[...] = jnp.full((_NT, 1), jnp.inf, dtype=jnp.float32)

        x = x_ref[...]
        ct = c_ref[...]
        dots = jnp.dot(x, ct, preferred_element_type=jnp.float32)
        c_sq = jnp.sum(ct * ct, axis=0, keepdims=True)
        part = c_sq - 2.0 * dots
        acc_ref[...] = jnp.minimum(acc_ref[...],
                                   jnp.min(part, axis=1, keepdims=True))

        @pl.when(j == nk - 1)
        def _finish():
            x_sq = jnp.sum(x * x, axis=1, keepdims=True)
            minv = acc_ref[...] + x_sq
            t_local = t0 + jax.lax.broadcasted_iota(jnp.int32, (_NT, 1), 0)
            masked = jnp.where(t_local < max_ilen, minv, 0.0)
            out_ref[0, 0] += jnp.sum(masked)


def kernel(xs, ilens, codebook):
    b, t, d = xs.shape
    k = codebook.shape[0]
    flat = xs.reshape(b * t, d)
    n = b * t
    c_t = codebook.T
    max_ilen = jnp.max(ilens)
    scalars = jnp.stack([max_ilen, jnp.int32(t)])

    total = pl.pallas_call(
        _vq_loss_kernel,
        grid=(n // _NT, k // _KT),
        in_specs=[
            pl.BlockSpec(memory_space=pltpu.SMEM),
            pl.BlockSpec((_NT, d), lambda i, j: (i, 0)),
            pl.BlockSpec((d, _KT), lambda i, j: (0, j)),
        ],
        out_specs=pl.BlockSpec((1, 1), lambda i, j: (0, 0),
                               memory_space=pltpu.SMEM),
        out_shape=jax.ShapeDtypeStruct((1, 1), jnp.float32),
        scratch_shapes=[pltpu.VMEM((_NT, 1), jnp.float32)],
        compiler_params=pltpu.CompilerParams(
            dimension_semantics=("arbitrary", "arbitrary")),
    )(scalars, flat, c_t)

    count = jnp.float32(b * d) * max_ilen.astype(jnp.float32)
    commit_loss = total[0, 0] / count
    loss = 0.25 * commit_loss
    return (loss, commit_loss)


# speedup vs baseline: 1.2607x; 1.1591x over previous
import jax
import jax.numpy as jnp
from jax.experimental import pallas as pl
from jax.experimental.pallas import tpu as pltpu

_NT = 512
_KT = 1024


def _vq_loss_kernel(maxlen_ref, x_ref, c_ref, out_ref, acc_ref):
    i = pl.program_id(0)
    j = pl.program_id(1)
    nk = pl.num_programs(1)
    max_ilen = maxlen_ref[0]
    t_dim = maxlen_ref[1]

    t0 = (i * _NT) % t_dim
    tile_active = t0 < max_ilen

    @pl.when(jnp.logical_and(i == 0, j == 0))
    def _init_out():
        out_ref[0, 0] = 0.0

    @pl.when(tile_active)
    def _compute():
        @pl.when(j == 0)
        def _init_acc():
            acc_ref[...] = jnp.full((_NT, 1), jnp.inf, dtype=jnp.float32)

        x = x_ref[...]
        ct = c_ref[...]
        dots = jnp.dot(x, ct, preferred_element_type=jnp.float32)
        c32 = ct.astype(jnp.float32)
        c_sq = jnp.sum(c32 * c32, axis=0, keepdims=True)
        part = c_sq - 2.0 * dots
        acc_ref[...] = jnp.minimum(acc_ref[...],
                                   jnp.min(part, axis=1, keepdims=True))

        @pl.when(j == nk - 1)
        def _finish():
            x32 = x.astype(jnp.float32)
            x_sq = jnp.sum(x32 * x32, axis=1, keepdims=True)
            minv = acc_ref[...] + x_sq
            t_local = t0 + jax.lax.broadcasted_iota(jnp.int32, (_NT, 1), 0)
            masked = jnp.where(t_local < max_ilen, minv, 0.0)
            out_ref[0, 0] += jnp.sum(masked)


def kernel(xs, ilens, codebook):
    b, t, d = xs.shape
    k = codebook.shape[0]
    flat = xs.reshape(b * t, d).astype(jnp.bfloat16)
    n = b * t
    c_t = codebook.T.astype(jnp.bfloat16)
    max_ilen = jnp.max(ilens)
    scalars = jnp.stack([max_ilen, jnp.int32(t)])

    total = pl.pallas_call(
        _vq_loss_kernel,
        grid=(n // _NT, k // _KT),
        in_specs=[
            pl.BlockSpec(memory_space=pltpu.SMEM),
            pl.BlockSpec((_NT, d), lambda i, j: (i, 0)),
            pl.BlockSpec((d, _KT), lambda i, j: (0, j)),
        ],
        out_specs=pl.BlockSpec((1, 1), lambda i, j: (0, 0),
                               memory_space=pltpu.SMEM),
        out_shape=jax.ShapeDtypeStruct((1, 1), jnp.float32),
        scratch_shapes=[pltpu.VMEM((_NT, 1), jnp.float32)],
        compiler_params=pltpu.CompilerParams(
            dimension_semantics=("arbitrary", "arbitrary")),
    )(scalars, flat, c_t)

    count = jnp.float32(b * d) * max_ilen.astype(jnp.float32)
    commit_loss = total[0, 0] / count
    loss = 0.25 * commit_loss
    return (loss, commit_loss)
